# baseline (speedup 1.0000x reference)
import jax
import jax.numpy as jnp
from jax import lax
from jax.experimental import pallas as pl
from jax.experimental.pallas import tpu as pltpu

B = 128
S = 25
N = S + 1
NN = N * N
EMBD = 128
LABD = 5
NB = 8
GRID = B // NB
F_OUT = (192, 192, 96, 96)

_f32 = jnp.float32
_bf16 = jnp.bfloat16


def _lrelu(v):
    return jnp.where(v >= 0, v, 0.01 * v)


def _accum_stats(stats_ref, h, ib):
    ps = jnp.sum(h, axis=0)[None, :]
    pss = jnp.sum(h * h, axis=0)[None, :]
    part = jnp.concatenate([ps, pss], axis=0)

    @pl.when(ib == 0)
    def _():
        stats_ref[...] = part

    @pl.when(ib > 0)
    def _():
        stats_ref[...] += part


def _layer0_tail(x, w_ref, b_ref, h_ref, stats_ref, ib):
    d = x.shape[-1]
    diff = jnp.abs(x[:, :, None, :] - x[:, None, :, :])
    a = diff.reshape(NB * NN, d).astype(_bf16)
    h = jnp.dot(a, w_ref[...], preferred_element_type=_f32) + b_ref[...]
    _accum_stats(stats_ref, h, ib)
    h_ref[...] = h.reshape(NB, N, N, F_OUT[0]).astype(_bf16)


def _passA0_kernel(z_ref, zi_ref, lab_ref, w_ref, b_ref,
                   h_ref, x_ref, stats_ref):
    ib = pl.program_id(0)
    z = z_ref[...]
    zi = zi_ref[...]
    labs = lab_ref[...]
    feats = jnp.concatenate([z[:, None, :], zi], axis=1)
    labs_f = jnp.concatenate(
        [jnp.zeros((NB, 1, LABD), _f32), labs], axis=1)
    x = jnp.concatenate([feats, labs_f], axis=2)
    x_ref[...] = x
    _layer0_tail(x, w_ref, b_ref, h_ref, stats_ref, ib)


def _passA_kernel(xp_ref, y_ref, st_ref, w_ref, b_ref,
                  h_ref, x_ref, stats_ref):
    ib = pl.program_id(0)
    xp = xp_ref[...]
    y = y_ref[...]
    s = st_ref[0, :]
    t = st_ref[1, :]
    xn = _lrelu(y * s + t)
    x = jnp.concatenate([xp, xn], axis=2)
    x_ref[...] = x
    _layer0_tail(x, w_ref, b_ref, h_ref, stats_ref, ib)


def _layer_kernel(hp_ref, st_ref, w_ref, b_ref, h_ref, stats_ref, *, fout):
    ib = pl.program_id(0)
    hp = hp_ref[...].astype(_f32)
    s = st_ref[0, :]
    t = st_ref[1, :]
    a = _lrelu(hp * s + t)
    fin = a.shape[-1]
    a2 = a.reshape(NB * NN, fin).astype(_bf16)
    h = jnp.dot(a2, w_ref[...], preferred_element_type=_f32) + b_ref[...]
    _accum_stats(stats_ref, h, ib)
    h_ref[...] = h.reshape(NB, N, N, fout).astype(_bf16)


def _adjacency(a, st_ref, wl_ref, bl_ref):
    s = st_ref[0, :]
    t = st_ref[1, :]
    a = _lrelu(a * s + t)
    wl = wl_ref[0, :]
    logit = jnp.sum(a * wl, axis=-1) + bl_ref[0, 0]
    ii = lax.broadcasted_iota(jnp.int32, logit.shape, logit.ndim - 2)
    jj = lax.broadcasted_iota(jnp.int32, logit.shape, logit.ndim - 1)
    logit = logit - jnp.where(ii == jj, 1e8, 0.0)
    logit = logit - jnp.max(logit, axis=-1, keepdims=True)
    e = jnp.exp(logit)
    return e / jnp.sum(e, axis=-1, keepdims=True)


def _passE_kernel(h3_ref, st_ref, wl_ref, bl_ref, x_ref, w1_ref, w2_ref,
                  bg_ref, y_ref, stats_ref):
    ib = pl.program_id(0)
    a = h3_ref[...].astype(_f32)
    adj = _adjacency(a, st_ref, wl_ref, bl_ref)
    xb = x_ref[...]
    d = xb.shape[-1]
    agg = lax.dot_general(adj, xb, (((2,), (1,)), ((0,), (0,))),
                          preferred_element_type=_f32)
    y = (jnp.dot(xb.reshape(NB * N, d), w1_ref[...],
                 preferred_element_type=_f32)
         + jnp.dot(agg.reshape(NB * N, d), w2_ref[...],
                   preferred_element_type=_f32)
         + bg_ref[...])
    _accum_stats(stats_ref, y, ib)
    y_ref[...] = y.reshape(NB, N, 48)


def _passEf_kernel(h3_ref, st_ref, wl_ref, bl_ref, x_ref, w1_ref, w2_ref,
                   bg_ref, sig_ref, ls_ref):
    a = h3_ref[...].astype(_f32)
    s = st_ref[0, :]
    t = st_ref[1, :]
    a = _lrelu(a * s + t)
    wl = wl_ref[0, :]
    logit = jnp.sum(a * wl, axis=-1) + bl_ref[0, 0]
    jj = lax.broadcasted_iota(jnp.int32, logit.shape, 2)
    logit = logit - jnp.where(jj == 0, 1e8, 0.0)
    logit = logit - jnp.max(logit, axis=-1, keepdims=True)
    e = jnp.exp(logit)
    adj0 = e / jnp.sum(e, axis=-1, keepdims=True)
    xb = x_ref[...]
    d = xb.shape[-1]
    agg = lax.dot_general(adj0, xb, (((2,), (1,)), ((0,), (0,))),
                          preferred_element_type=_f32)
    lg = (jnp.dot(xb[:, 0, :], w1_ref[...], preferred_element_type=_f32)
          + jnp.dot(agg.reshape(NB, d), w2_ref[...],
                    preferred_element_type=_f32)
          + bg_ref[...])
    sig_ref[...] = 1.0 / (1.0 + jnp.exp(-lg))
    m = jnp.max(lg, axis=1, keepdims=True)
    ls_ref[...] = lg - (m + jnp.log(jnp.sum(jnp.exp(lg - m), axis=1,
                                            keepdims=True)))


def _seq_params():
    return pltpu.CompilerParams(dimension_semantics=("arbitrary",))


def _full_spec(shape):
    return pl.BlockSpec(shape, lambda ib: tuple(0 for _ in shape))


def _blk_spec(shape):
    return pl.BlockSpec(shape, lambda ib: (ib,) + tuple(0 for _ in shape[1:]))


def _fold_bn(stats, g, bt, count):
    mean = stats[0] / count
    var = stats[1] / count - mean * mean
    s = g * lax.rsqrt(var + 1e-5)
    return jnp.stack([s, bt - mean * s], axis=0)


def _wc_weights(p, d_in):
    ws = [p['w%d' % i].T.astype(_bf16) for i in range(4)]
    bs = [p['b%d' % i][None, :] for i in range(4)]
    return ws, bs


def _run_layers(h0, stats0, wp):
    h = h0
    stats = stats0
    folds = []
    for k in (1, 2, 3):
        fold = _fold_bn(stats, wp['g%d' % (k - 1)], wp['bt%d' % (k - 1)],
                        float(B * NN))
        folds.append(fold)
        fin = F_OUT[k - 1]
        fout = F_OUT[k]
        w = wp['w%d' % k].T.astype(_bf16)
        b = wp['b%d' % k][None, :]
        h, stats = pl.pallas_call(
            lambda hp, st, wr, br, hr, sr, _fo=fout: _layer_kernel(
                hp, st, wr, br, hr, sr, fout=_fo),
            grid=(GRID,),
            in_specs=[_blk_spec((NB, N, N, fin)),
                      _full_spec((2, fin)),
                      _full_spec((fin, fout)),
                      _full_spec((1, fout))],
            out_specs=[_blk_spec((NB, N, N, fout)),
                       _full_spec((2, fout))],
            out_shape=[jax.ShapeDtypeStruct((B, N, N, fout), _bf16),
                       jax.ShapeDtypeStruct((2, fout), _f32)],
            compiler_params=_seq_params(),
        )(h, fold, w, b)
    fold3 = _fold_bn(stats, wp['g3'], wp['bt3'], float(B * NN))
    return h, fold3


def _gc_weights(gp, d):
    w1 = gp['fc_w'][:, :d].T
    w2 = gp['fc_w'][:, d:].T
    return w1, w2, gp['fc_b'][None, :]


def kernel(z, zi_s, labels_yi, params):
    zi_t = jnp.transpose(zi_s, (1, 0, 2))
    lab_t = jnp.transpose(labels_yi, (1, 0, 2))

    dims = (EMBD + LABD, EMBD + LABD + 48, EMBD + LABD + 96)
    x = None
    y_raw = None
    y_fold = None
    for r in range(3):
        wp = params['wc%d' % r] if r < 2 else params['wcl']
        gp = params['gc%d' % r] if r < 2 else params['gcl']
        d = dims[r]
        w0 = wp['w0'].T.astype(_bf16)
        b0 = wp['b0'][None, :]
        if r == 0:
            h0, x, stats0 = pl.pallas_call(
                _passA0_kernel,
                grid=(GRID,),
                in_specs=[_blk_spec((NB, EMBD)),
                          _blk_spec((NB, S, EMBD)),
                          _blk_spec((NB, S, LABD)),
                          _full_spec((d, F_OUT[0])),
                          _full_spec((1, F_OUT[0]))],
                out_specs=[_blk_spec((NB, N, N, F_OUT[0])),
                           _blk_spec((NB, N, d)),
                           _full_spec((2, F_OUT[0]))],
                out_shape=[jax.ShapeDtypeStruct((B, N, N, F_OUT[0]), _bf16),
                           jax.ShapeDtypeStruct((B, N, d), _f32),
                           jax.ShapeDtypeStruct((2, F_OUT[0]), _f32)],
                compiler_params=_seq_params(),
            )(z, zi_t, lab_t, w0, b0)
        else:
            d_prev = dims[r - 1]
            h0, x, stats0 = pl.pallas_call(
                _passA_kernel,
                grid=(GRID,),
                in_specs=[_blk_spec((NB, N, d_prev)),
                          _blk_spec((NB, N, 48)),
                          _full_spec((2, 48)),
                          _full_spec((d, F_OUT[0])),
                          _full_spec((1, F_OUT[0]))],
                out_specs=[_blk_spec((NB, N, N, F_OUT[0])),
                           _blk_spec((NB, N, d)),
                           _full_spec((2, F_OUT[0]))],
                out_shape=[jax.ShapeDtypeStruct((B, N, N, F_OUT[0]), _bf16),
                           jax.ShapeDtypeStruct((B, N, d), _f32),
                           jax.ShapeDtypeStruct((2, F_OUT[0]), _f32)],
                compiler_params=_seq_params(),
            )(x, y_raw, y_fold, w0, b0)

        h3, fold3 = _run_layers(h0, stats0, wp)
        wl = wp['wl']
        bl = wp['bl'][None, :]
        w1g, w2g, bg = _gc_weights(gp, d)

        if r < 2:
            y_raw, y_stats = pl.pallas_call(
                _passE_kernel,
                grid=(GRID,),
                in_specs=[_blk_spec((NB, N, N, F_OUT[3])),
                          _full_spec((2, F_OUT[3])),
                          _full_spec((1, F_OUT[3])),
                          _full_spec((1, 1)),
                          _blk_spec((NB, N, d)),
                          _full_spec((d, 48)),
                          _full_spec((d, 48)),
                          _full_spec((1, 48))],
                out_specs=[_blk_spec((NB, N, 48)),
                           _full_spec((2, 48))],
                out_shape=[jax.ShapeDtypeStruct((B, N, 48), _f32),
                           jax.ShapeDtypeStruct((2, 48), _f32)],
                compiler_params=_seq_params(),
            )(h3, fold3, wl, bl, x, w1g, w2g, bg)
            y_fold = _fold_bn(y_stats, gp['g'], gp['bt'], float(B * N))
        else:
            sig, ls = pl.pallas_call(
                _passEf_kernel,
                grid=(GRID,),
                in_specs=[pl.BlockSpec((NB, 1, N, F_OUT[3]),
                                       lambda ib: (ib, 0, 0, 0)),
                          _full_spec((2, F_OUT[3])),
                          _full_spec((1, F_OUT[3])),
                          _full_spec((1, 1)),
                          _blk_spec((NB, N, d)),
                          _full_spec((d, LABD)),
                          _full_spec((d, LABD)),
                          _full_spec((1, LABD))],
                out_specs=[_blk_spec((NB, LABD)),
                           _blk_spec((NB, LABD))],
                out_shape=[jax.ShapeDtypeStruct((B, LABD), _f32),
                           jax.ShapeDtypeStruct((B, LABD), _f32)],
                compiler_params=_seq_params(),
            )(h3, fold3, wl, bl, x, w1g, w2g, bg)
            return sig, ls


# speedup vs baseline: 1.2237x; 1.2237x over previous
import jax
import jax.numpy as jnp
from jax import lax
from jax.experimental import pallas as pl
from jax.experimental.pallas import tpu as pltpu

B = 128
S = 25
N = S + 1
NN = N * N
P_PIX = B * NN
EMBD = 128
LABD = 5
NB = 16
GRID = B // NB
RB = NB * NN
F_OUT = (192, 192, 96, 96)

_f32 = jnp.float32
_bf16 = jnp.bfloat16


def _lrelu(v):
    return jnp.maximum(v, 0.01 * v)


def _accum_stats(stats_ref, hb, ib):
    ones = jnp.ones((1, hb.shape[0]), _bf16)
    ps = jnp.dot(ones, hb, preferred_element_type=_f32)
    pss = jnp.dot(ones, (hb * hb).astype(_bf16), preferred_element_type=_f32)
    part = jnp.concatenate([ps, pss], axis=0)

    @pl.when(ib == 0)
    def _():
        stats_ref[...] = part

    @pl.when(ib > 0)
    def _():
        stats_ref[...] += part


def _layer0_tail(x, w_ref, b_ref, h_ref, stats_ref, ib):
    d = x.shape[-1]
    xb = x.astype(_bf16)
    diff = jnp.abs(xb[:, :, None, :] - xb[:, None, :, :])
    a = diff.reshape(RB, d)
    h = jnp.dot(a, w_ref[...], preferred_element_type=_f32) + b_ref[...]
    hb = h.astype(_bf16)
    _accum_stats(stats_ref, hb, ib)
    h_ref[...] = hb


def _passA0_kernel(z_ref, zi_ref, lab_ref, w_ref, b_ref,
                   h_ref, x_ref, stats_ref):
    ib = pl.program_id(0)
    z = z_ref[...]
    zi = zi_ref[...]
    labs = lab_ref[...]
    feats = jnp.concatenate([z[:, None, :], zi], axis=1)
    labs_f = jnp.concatenate(
        [jnp.zeros((NB, 1, LABD), _f32), labs], axis=1)
    x = jnp.concatenate([feats, labs_f], axis=2)
    x_ref[...] = x
    _layer0_tail(x, w_ref, b_ref, h_ref, stats_ref, ib)


def _passA_kernel(xp_ref, y_ref, st_ref, w_ref, b_ref,
                  h_ref, x_ref, stats_ref):
    ib = pl.program_id(0)
    xp = xp_ref[...]
    y = y_ref[...]
    s = st_ref[0, :]
    t = st_ref[1, :]
    xn = _lrelu(y * s + t)
    x = jnp.concatenate([xp, xn], axis=2)
    x_ref[...] = x
    _layer0_tail(x, w_ref, b_ref, h_ref, stats_ref, ib)


def _query_rows(hb):
    return jnp.stack([hb[e * NN:e * NN + N, :] for e in range(NB)], axis=0)


def _layer_kernel(hp_ref, t_ref, w_ref, b_ref, h_ref, stats_ref, *q_ref):
    ib = pl.program_id(0)
    a = _lrelu(hp_ref[...] + t_ref[...])
    h = jnp.dot(a, w_ref[...], preferred_element_type=_f32) + b_ref[...]
    hb = h.astype(_bf16)
    _accum_stats(stats_ref, hb, ib)
    h_ref[...] = hb
    if q_ref:
        q_ref[0][...] = _query_rows(hb)


def _passE_kernel(h3_ref, t3_ref, wl_ref, x_ref, w1_ref, w2_ref,
                  bg_ref, y_ref, stats_ref):
    ib = pl.program_id(0)
    a = _lrelu(h3_ref[...] + t3_ref[...])
    lg = jnp.dot(a, wl_ref[...], preferred_element_type=_f32)
    logit = lg[:, :1].reshape(NB * N, N)
    ii = lax.broadcasted_iota(jnp.int32, logit.shape, 0)
    jj = lax.broadcasted_iota(jnp.int32, logit.shape, 1)
    logit = logit - jnp.where(ii % N == jj, 1e8, 0.0)
    logit = logit - jnp.max(logit, axis=-1, keepdims=True)
    e = jnp.exp(logit)
    adj = (e / jnp.sum(e, axis=-1, keepdims=True)).reshape(NB, N, N)
    xb = x_ref[...]
    d = xb.shape[-1]
    agg = lax.dot_general(adj, xb, (((2,), (1,)), ((0,), (0,))),
                          preferred_element_type=_f32)
    y = (jnp.dot(xb.reshape(NB * N, d), w1_ref[...],
                 preferred_element_type=_f32)
         + jnp.dot(agg.reshape(NB * N, d), w2_ref[...],
                   preferred_element_type=_f32)
         + bg_ref[...])
    ones = jnp.ones((1, NB * N), _f32)
    ps = jnp.dot(ones, y, preferred_element_type=_f32)
    pss = jnp.dot(ones, y * y, preferred_element_type=_f32)
    part = jnp.concatenate([ps, pss], axis=0)

    @pl.when(ib == 0)
    def _():
        stats_ref[...] = part

    @pl.when(ib > 0)
    def _():
        stats_ref[...] += part

    y_ref[...] = y.reshape(NB, N, 48)


def _passEf_kernel(q_ref, st_ref, wl_ref, bl_ref, x_ref, w1_ref, w2_ref,
                   bg_ref, sig_ref, ls_ref):
    a = q_ref[...].astype(_f32)
    s = st_ref[0, :]
    t = st_ref[1, :]
    a = _lrelu(a * s + t)
    wl = wl_ref[0, :]
    logit = jnp.sum(a * wl, axis=-1) + bl_ref[0, 0]
    jj = lax.broadcasted_iota(jnp.int32, logit.shape, 1)
    logit = logit - jnp.where(jj == 0, 1e8, 0.0)
    logit = logit - jnp.max(logit, axis=-1, keepdims=True)
    e = jnp.exp(logit)
    adj0 = (e / jnp.sum(e, axis=-1, keepdims=True))[:, None, :]
    xb = x_ref[...]
    d = xb.shape[-1]
    agg = lax.dot_general(adj0, xb, (((2,), (1,)), ((0,), (0,))),
                          preferred_element_type=_f32)
    lg = (jnp.dot(xb[:, 0, :], w1_ref[...], preferred_element_type=_f32)
          + jnp.dot(agg.reshape(NB, d), w2_ref[...],
                    preferred_element_type=_f32)
          + bg_ref[...])
    sig_ref[...] = 1.0 / (1.0 + jnp.exp(-lg))
    m = jnp.max(lg, axis=1, keepdims=True)
    ls_ref[...] = lg - (m + jnp.log(jnp.sum(jnp.exp(lg - m), axis=1,
                                            keepdims=True)))


def _seq_params():
    return pltpu.CompilerParams(dimension_semantics=("arbitrary",))


def _full_spec(shape):
    return pl.BlockSpec(shape, lambda ib: tuple(0 for _ in shape))


def _blk_spec(shape):
    return pl.BlockSpec(shape, lambda ib: (ib,) + tuple(0 for _ in shape[1:]))


def _bn_fold(stats, g, bt, count):
    mean = stats[0] / count
    var = stats[1] / count - mean * mean
    s = g * lax.rsqrt(var + 1e-5)
    return s, bt - mean * s


def _run_layers(h0, stats0, wp):
    h = h0
    stats = stats0
    q = None
    for k in (1, 2, 3):
        s, t = _bn_fold(stats, wp['g%d' % (k - 1)], wp['bt%d' % (k - 1)],
                        float(P_PIX))
        fin = F_OUT[k - 1]
        fout = F_OUT[k]
        w = (wp['w%d' % k] * s[None, :]).T.astype(_bf16)
        tk = (t / s)[None, :].astype(_bf16)
        b = wp['b%d' % k][None, :]
        last = k == 3
        out_specs = [_blk_spec((RB, fout)), _full_spec((2, fout))]
        out_shape = [jax.ShapeDtypeStruct((P_PIX, fout), _bf16),
                     jax.ShapeDtypeStruct((2, fout), _f32)]
        if last:
            out_specs.append(_blk_spec((NB, N, fout)))
            out_shape.append(jax.ShapeDtypeStruct((B, N, fout), _bf16))
        res = pl.pallas_call(
            _layer_kernel,
            grid=(GRID,),
            in_specs=[_blk_spec((RB, fin)),
                      _full_spec((1, fin)),
                      _full_spec((fin, fout)),
                      _full_spec((1, fout))],
            out_specs=out_specs,
            out_shape=out_shape,
            compiler_params=_seq_params(),
        )(h, tk, w, b)
        h, stats = res[0], res[1]
        if last:
            q = res[2]
    s3, t3 = _bn_fold(stats, wp['g3'], wp['bt3'], float(P_PIX))
    return h, q, s3, t3


def _gc_weights(gp, d):
    w1 = gp['fc_w'][:, :d].T
    w2 = gp['fc_w'][:, d:].T
    return w1, w2, gp['fc_b'][None, :]


def kernel(z, zi_s, labels_yi, params):
    zi_t = jnp.transpose(zi_s, (1, 0, 2))
    lab_t = jnp.transpose(labels_yi, (1, 0, 2))

    dims = (EMBD + LABD, EMBD + LABD + 48, EMBD + LABD + 96)
    x = None
    y_raw = None
    y_fold = None
    for r in range(3):
        wp = params['wc%d' % r] if r < 2 else params['wcl']
        gp = params['gc%d' % r] if r < 2 else params['gcl']
        d = dims[r]
        w0 = wp['w0'].T.astype(_bf16)
        b0 = wp['b0'][None, :]
        a_outs = [jax.ShapeDtypeStruct((P_PIX, F_OUT[0]), _bf16),
                  jax.ShapeDtypeStruct((B, N, d), _f32),
                  jax.ShapeDtypeStruct((2, F_OUT[0]), _f32)]
        a_ospecs = [_blk_spec((RB, F_OUT[0])),
                    _blk_spec((NB, N, d)),
                    _full_spec((2, F_OUT[0]))]
        if r == 0:
            h0, x, stats0 = pl.pallas_call(
                _passA0_kernel,
                grid=(GRID,),
                in_specs=[_blk_spec((NB, EMBD)),
                          _blk_spec((NB, S, EMBD)),
                          _blk_spec((NB, S, LABD)),
                          _full_spec((d, F_OUT[0])),
                          _full_spec((1, F_OUT[0]))],
                out_specs=a_ospecs,
                out_shape=a_outs,
                compiler_params=_seq_params(),
            )(z, zi_t, lab_t, w0, b0)
        else:
            d_prev = dims[r - 1]
            h0, x, stats0 = pl.pallas_call(
                _passA_kernel,
                grid=(GRID,),
                in_specs=[_blk_spec((NB, N, d_prev)),
                          _blk_spec((NB, N, 48)),
                          _full_spec((2, 48)),
                          _full_spec((d, F_OUT[0])),
                          _full_spec((1, F_OUT[0]))],
                out_specs=a_ospecs,
                out_shape=a_outs,
                compiler_params=_seq_params(),
            )(x, y_raw, y_fold, w0, b0)

        h3, q3, s3, t3 = _run_layers(h0, stats0, wp)
        w1g, w2g, bg = _gc_weights(gp, d)

        if r < 2:
            wl_col = (wp['wl'][0] * s3)[:, None]
            wl_mat = jnp.broadcast_to(wl_col, (96, 128)).astype(_bf16)
            t3k = (t3 / s3)[None, :].astype(_bf16)
            y_raw, y_stats = pl.pallas_call(
                _passE_kernel,
                grid=(GRID,),
                in_specs=[_blk_spec((RB, F_OUT[3])),
                          _full_spec((1, F_OUT[3])),
                          _full_spec((F_OUT[3], 128)),
                          _blk_spec((NB, N, d)),
                          _full_spec((d, 48)),
                          _full_spec((d, 48)),
                          _full_spec((1, 48))],
                out_specs=[_blk_spec((NB, N, 48)),
                           _full_spec((2, 48))],
                out_shape=[jax.ShapeDtypeStruct((B, N, 48), _f32),
                           jax.ShapeDtypeStruct((2, 48), _f32)],
                compiler_params=_seq_params(),
            )(h3, t3k, wl_mat, x, w1g, w2g, bg)
            ys, yt = _bn_fold(y_stats, gp['g'], gp['bt'], float(B * N))
            y_fold = jnp.stack([ys, yt], axis=0)
        else:
            fold3 = jnp.stack([s3, t3], axis=0)
            sig, ls = pl.pallas_call(
                _passEf_kernel,
                grid=(GRID,),
                in_specs=[_blk_spec((NB, N, F_OUT[3])),
                          _full_spec((2, F_OUT[3])),
                          _full_spec((1, F_OUT[3])),
                          _full_spec((1, 1)),
                          _blk_spec((NB, N, d)),
                          _full_spec((d, LABD)),
                          _full_spec((d, LABD)),
                          _full_spec((1, LABD))],
                out_specs=[_blk_spec((NB, LABD)),
                           _blk_spec((NB, LABD))],
                out_shape=[jax.ShapeDtypeStruct((B, LABD), _f32),
                           jax.ShapeDtypeStruct((B, LABD), _f32)],
                compiler_params=_seq_params(),
            )(q3, fold3, wp['wl'], wp['bl'][None, :], x, w1g, w2g, bg)
            return sig, ls


# speedup vs baseline: 2.2103x; 1.8063x over previous
import numpy as np
import jax
import jax.numpy as jnp
from jax import lax
from jax.experimental import pallas as pl
from jax.experimental.pallas import tpu as pltpu

B = 128
S = 25
N = S + 1
NN = N * N
P_PIX = B * NN
EMBD = 128
LABD = 5
NK = 13
KP = 32
PPE = NK * KP
NB = 16
GRID = B // NB
RB = NB * PPE
F_OUT = (192, 192, 96, 96)

_f32 = jnp.float32
_bf16 = jnp.bfloat16


def _pair_index_tables():
    idx = np.zeros((N, N), np.int32)
    for i in range(N):
        for j in range(N):
            if i == j:
                idx[i, j] = N
                continue
            k = (j - i) % N
            if k <= NK:
                idx[i, j] = (k - 1) * KP + i
            else:
                k2 = (i - j) % N
                idx[i, j] = (k2 - 1) * KP + j
    w = np.zeros((PPE, 1), np.float32)
    for kb in range(NK):
        w[kb * KP:kb * KP + N, 0] = 2.0 if kb < NK - 1 else 1.0
        w[kb * KP + N:(kb + 1) * KP, 0] = float(N) / float(NK * (KP - N))
    return idx.reshape(-1), w


_PAIR_IDX, _ROW_W = _pair_index_tables()


def _lrelu(v):
    return jnp.maximum(v, 0.01 * v)


def _accum_stats(stats_ref, h, w, ib):
    wh = h * w
    ps = jnp.sum(wh, axis=0)[None, :]
    pss = jnp.sum(wh * h, axis=0)[None, :]
    part = jnp.concatenate([ps, pss], axis=0)

    @pl.when(ib == 0)
    def _():
        stats_ref[...] = part

    @pl.when(ib > 0)
    def _():
        stats_ref[...] += part


def _layer0_tail(x, w_ref, b_ref, rw_ref, h_ref, stats_ref, ib):
    d = x.shape[-1]
    xb = x.astype(_bf16)
    zpad = jnp.zeros((NB, KP - N, d), _bf16)
    xp = jnp.concatenate([xb, zpad], axis=1)
    blocks = []
    for k in range(1, NK + 1):
        shifted = jnp.concatenate([xb[:, k:, :], xb[:, :k, :], zpad], axis=1)
        blocks.append(jnp.abs(xp - shifted))
    a = jnp.concatenate(blocks, axis=1).reshape(RB, d)
    h = jnp.dot(a, w_ref[...], preferred_element_type=_f32) + b_ref[...]
    _accum_stats(stats_ref, h, rw_ref[...], ib)
    h_ref[...] = h.astype(_bf16)


def _passA0_kernel(z_ref, zi_ref, lab_ref, w_ref, b_ref, rw_ref,
                   h_ref, x_ref, stats_ref):
    ib = pl.program_id(0)
    z = z_ref[...]
    zi = zi_ref[...]
    labs = lab_ref[...]
    feats = jnp.concatenate([z[:, None, :], zi], axis=1)
    labs_f = jnp.concatenate(
        [jnp.zeros((NB, 1, LABD), _f32), labs], axis=1)
    x = jnp.concatenate([feats, labs_f], axis=2)
    x_ref[...] = x
    _layer0_tail(x, w_ref, b_ref, rw_ref, h_ref, stats_ref, ib)


def _passA_kernel(xp_ref, y_ref, st_ref, w_ref, b_ref, rw_ref,
                  h_ref, x_ref, stats_ref):
    ib = pl.program_id(0)
    xp = xp_ref[...]
    y = y_ref[...]
    s = st_ref[0, :]
    t = st_ref[1, :]
    xn = _lrelu(y * s + t)
    x = jnp.concatenate([xp, xn], axis=2)
    x_ref[...] = x
    _layer0_tail(x, w_ref, b_ref, rw_ref, h_ref, stats_ref, ib)


def _layer_kernel(hp_ref, t_ref, w_ref, b_ref, rw_ref, h_ref, stats_ref):
    ib = pl.program_id(0)
    a = _lrelu(hp_ref[...] + t_ref[...])
    h = jnp.dot(a, w_ref[...], preferred_element_type=_f32) + b_ref[...]
    _accum_stats(stats_ref, h, rw_ref[...], ib)
    h_ref[...] = h.astype(_bf16)


def _proj_kernel(h3_ref, t3_ref, wl_ref, l_ref):
    a = _lrelu(h3_ref[...] + t3_ref[...])
    l_ref[...] = jnp.dot(a, wl_ref[...], preferred_element_type=_f32)


def _passE_kernel(lm_ref, x_ref, w1_ref, w2_ref, bg_ref, y_ref, stats_ref):
    ib = pl.program_id(0)
    logit = lm_ref[...]
    ii = lax.broadcasted_iota(jnp.int32, logit.shape, 1)
    jj = lax.broadcasted_iota(jnp.int32, logit.shape, 2)
    logit = logit - jnp.where(ii == jj, 1e8, 0.0)
    logit = logit - jnp.max(logit, axis=-1, keepdims=True)
    e = jnp.exp(logit)
    adj = e / jnp.sum(e, axis=-1, keepdims=True)
    xb = x_ref[...]
    d = xb.shape[-1]
    agg = lax.dot_general(adj, xb, (((2,), (1,)), ((0,), (0,))),
                          preferred_element_type=_f32)
    y = (jnp.dot(xb.reshape(NB * N, d), w1_ref[...],
                 preferred_element_type=_f32)
         + jnp.dot(agg.reshape(NB * N, d), w2_ref[...],
                   preferred_element_type=_f32)
         + bg_ref[...])
    ones = jnp.ones((NB * N, 1), _f32)
    _accum_stats(stats_ref, y, ones, ib)
    y_ref[...] = y.reshape(NB, N, 48)


def _passEf_kernel(l0_ref, x_ref, w1_ref, w2_ref, bg_ref, sig_ref, ls_ref):
    logit = l0_ref[...]
    jj = lax.broadcasted_iota(jnp.int32, logit.shape, 1)
    logit = logit - jnp.where(jj == 0, 1e8, 0.0)
    logit = logit - jnp.max(logit, axis=-1, keepdims=True)
    e = jnp.exp(logit)
    adj0 = (e / jnp.sum(e, axis=-1, keepdims=True))[:, None, :]
    xb = x_ref[...]
    d = xb.shape[-1]
    agg = lax.dot_general(adj0, xb, (((2,), (1,)), ((0,), (0,))),
                          preferred_element_type=_f32)
    lg = (jnp.dot(xb[:, 0, :], w1_ref[...], preferred_element_type=_f32)
          + jnp.dot(agg.reshape(NB, d), w2_ref[...],
                    preferred_element_type=_f32)
          + bg_ref[...])
    sig_ref[...] = 1.0 / (1.0 + jnp.exp(-lg))
    m = jnp.max(lg, axis=1, keepdims=True)
    ls_ref[...] = lg - (m + jnp.log(jnp.sum(jnp.exp(lg - m), axis=1,
                                            keepdims=True)))


def _seq_params():
    return pltpu.CompilerParams(dimension_semantics=("arbitrary",))


def _full_spec(shape):
    return pl.BlockSpec(shape, lambda ib: tuple(0 for _ in shape))


def _blk_spec(shape):
    return pl.BlockSpec(shape, lambda ib: (ib,) + tuple(0 for _ in shape[1:]))


def _bn_fold(stats, g, bt, count):
    mean = stats[0] / count
    var = stats[1] / count - mean * mean
    s = g * lax.rsqrt(var + 1e-5)
    return s, bt - mean * s


def _run_layers(h0, stats0, wp, row_w):
    h = h0
    stats = stats0
    for k in (1, 2, 3):
        s, t = _bn_fold(stats, wp['g%d' % (k - 1)], wp['bt%d' % (k - 1)],
                        float(P_PIX))
        fin = F_OUT[k - 1]
        fout = F_OUT[k]
        w = (wp['w%d' % k] * s[None, :]).T.astype(_bf16)
        tk = (t / s)[None, :].astype(_bf16)
        b = wp['b%d' % k][None, :]
        h, stats = pl.pallas_call(
            _layer_kernel,
            grid=(GRID,),
            in_specs=[_blk_spec((RB, fin)),
                      _full_spec((1, fin)),
                      _full_spec((fin, fout)),
                      _full_spec((1, fout)),
                      _full_spec((RB, 1))],
            out_specs=[_blk_spec((RB, fout)), _full_spec((2, fout))],
            out_shape=[jax.ShapeDtypeStruct((B * PPE, fout), _bf16),
                       jax.ShapeDtypeStruct((2, fout), _f32)],
            compiler_params=_seq_params(),
        )(h, tk, w, b, row_w)
    s3, t3 = _bn_fold(stats, wp['g3'], wp['bt3'], float(P_PIX))
    return h, s3, t3


def _gc_weights(gp, d):
    w1 = gp['fc_w'][:, :d].T
    w2 = gp['fc_w'][:, d:].T
    return w1, w2, gp['fc_b'][None, :]


def kernel(z, zi_s, labels_yi, params):
    zi_t = jnp.transpose(zi_s, (1, 0, 2))
    lab_t = jnp.transpose(labels_yi, (1, 0, 2))
    row_w = jnp.asarray(np.tile(_ROW_W, (NB, 1)))
    pair_idx = jnp.asarray(_PAIR_IDX)

    dims = (EMBD + LABD, EMBD + LABD + 48, EMBD + LABD + 96)
    x = None
    y_raw = None
    y_fold = None
    for r in range(3):
        wp = params['wc%d' % r] if r < 2 else params['wcl']
        gp = params['gc%d' % r] if r < 2 else params['gcl']
        d = dims[r]
        w0 = wp['w0'].T.astype(_bf16)
        b0 = wp['b0'][None, :]
        a_outs = [jax.ShapeDtypeStruct((B * PPE, F_OUT[0]), _bf16),
                  jax.ShapeDtypeStruct((B, N, d), _f32),
                  jax.ShapeDtypeStruct((2, F_OUT[0]), _f32)]
        a_ospecs = [_blk_spec((RB, F_OUT[0])),
                    _blk_spec((NB, N, d)),
                    _full_spec((2, F_OUT[0]))]
        if r == 0:
            h0, x, stats0 = pl.pallas_call(
                _passA0_kernel,
                grid=(GRID,),
                in_specs=[_blk_spec((NB, EMBD)),
                          _blk_spec((NB, S, EMBD)),
                          _blk_spec((NB, S, LABD)),
                          _full_spec((d, F_OUT[0])),
                          _full_spec((1, F_OUT[0])),
                          _full_spec((RB, 1))],
                out_specs=a_ospecs,
                out_shape=a_outs,
                compiler_params=_seq_params(),
            )(z, zi_t, lab_t, w0, b0, row_w)
        else:
            d_prev = dims[r - 1]
            h0, x, stats0 = pl.pallas_call(
                _passA_kernel,
                grid=(GRID,),
                in_specs=[_blk_spec((NB, N, d_prev)),
                          _blk_spec((NB, N, 48)),
                          _full_spec((2, 48)),
                          _full_spec((d, F_OUT[0])),
                          _full_spec((1, F_OUT[0])),
                          _full_spec((RB, 1))],
                out_specs=a_ospecs,
                out_shape=a_outs,
                compiler_params=_seq_params(),
            )(x, y_raw, y_fold, w0, b0, row_w)

        h3, s3, t3 = _run_layers(h0, stats0, wp, row_w)
        w1g, w2g, bg = _gc_weights(gp, d)

        wl_col = (wp['wl'][0] * s3)[:, None].astype(_bf16)
        t3k = (t3 / s3)[None, :].astype(_bf16)
        l_col = pl.pallas_call(
            _proj_kernel,
            grid=(GRID,),
            in_specs=[_blk_spec((RB, F_OUT[3])),
                      _full_spec((1, F_OUT[3])),
                      _full_spec((F_OUT[3], 1))],
            out_specs=_blk_spec((RB, 1)),
            out_shape=jax.ShapeDtypeStruct((B * PPE, 1), _f32),
            compiler_params=_seq_params(),
        )(h3, t3k, wl_col)
        l_pk = l_col.reshape(B, PPE)

        if r < 2:
            l_mat = jnp.take(l_pk, pair_idx, axis=1).reshape(B, N, N)
            y_raw, y_stats = pl.pallas_call(
                _passE_kernel,
                grid=(GRID,),
                in_specs=[_blk_spec((NB, N, N)),
                          _blk_spec((NB, N, d)),
                          _full_spec((d, 48)),
                          _full_spec((d, 48)),
                          _full_spec((1, 48))],
                out_specs=[_blk_spec((NB, N, 48)),
                           _full_spec((2, 48))],
                out_shape=[jax.ShapeDtypeStruct((B, N, 48), _f32),
                           jax.ShapeDtypeStruct((2, 48), _f32)],
                compiler_params=_seq_params(),
            )(l_mat, x, w1g, w2g, bg)
            ys, yt = _bn_fold(y_stats, gp['g'], gp['bt'], float(B * N))
            y_fold = jnp.stack([ys, yt], axis=0)
        else:
            l0 = jnp.take(l_pk, pair_idx[:N], axis=1)
            sig, ls = pl.pallas_call(
                _passEf_kernel,
                grid=(GRID,),
                in_specs=[_blk_spec((NB, N)),
                          _blk_spec((NB, N, d)),
                          _full_spec((d, LABD)),
                          _full_spec((d, LABD)),
                          _full_spec((1, LABD))],
                out_specs=[_blk_spec((NB, LABD)),
                           _blk_spec((NB, LABD))],
                out_shape=[jax.ShapeDtypeStruct((B, LABD), _f32),
                           jax.ShapeDtypeStruct((B, LABD), _f32)],
                compiler_params=_seq_params(),
            )(l0, x, w1g, w2g, bg)
            return sig, ls


# speedup vs baseline: 2.2215x; 1.0051x over previous
import numpy as np
import jax
import jax.numpy as jnp
from jax import lax
from jax.experimental import pallas as pl
from jax.experimental.pallas import tpu as pltpu

B = 128
S = 25
N = S + 1
NN = N * N
P_PIX = B * NN
EMBD = 128
LABD = 5
NK = 13
KP = 32
PPE = NK * KP
NB = 32
GRID = B // NB
RB = NB * PPE
F_OUT = (192, 192, 96, 96)

_f32 = jnp.float32
_bf16 = jnp.bfloat16


def _pair_index_tables():
    idx = np.zeros((N, N), np.int32)
    for i in range(N):
        for j in range(N):
            if i == j:
                idx[i, j] = N
                continue
            k = (j - i) % N
            if k <= NK:
                idx[i, j] = (k - 1) * KP + i
            else:
                k2 = (i - j) % N
                idx[i, j] = (k2 - 1) * KP + j
    w = np.zeros((PPE, 1), np.float32)
    for kb in range(NK):
        w[kb * KP:kb * KP + N, 0] = 2.0 if kb < NK - 1 else 1.0
        w[kb * KP + N:(kb + 1) * KP, 0] = float(N) / float(NK * (KP - N))
    return idx.reshape(-1), w


_PAIR_IDX, _ROW_W = _pair_index_tables()


def _lrelu(v):
    return jnp.maximum(v, 0.01 * v)


def _accum_stats(stats_ref, h, w, ib):
    wh = h * w
    ps = jnp.sum(wh, axis=0)[None, :]
    pss = jnp.sum(wh * h, axis=0)[None, :]
    part = jnp.concatenate([ps, pss], axis=0)

    @pl.when(ib == 0)
    def _():
        stats_ref[...] = part

    @pl.when(ib > 0)
    def _():
        stats_ref[...] += part


def _layer0_tail(x, w_ref, b_ref, rw_ref, h_ref, stats_ref, ib):
    d = x.shape[-1]
    xb = x.astype(_bf16)
    zpad = jnp.zeros((NB, KP - N, d), _bf16)
    xp = jnp.concatenate([xb, zpad], axis=1)
    blocks = []
    for k in range(1, NK + 1):
        shifted = jnp.concatenate([xb[:, k:, :], xb[:, :k, :], zpad], axis=1)
        blocks.append(jnp.abs(xp - shifted))
    a = jnp.concatenate(blocks, axis=1).reshape(RB, d)
    h = jnp.dot(a, w_ref[...], preferred_element_type=_f32) + b_ref[...]
    _accum_stats(stats_ref, h, rw_ref[...], ib)
    h_ref[...] = h.astype(_bf16)


def _passA0_kernel(z_ref, zi_ref, lab_ref, w_ref, b_ref, rw_ref,
                   h_ref, x_ref, stats_ref):
    ib = pl.program_id(0)
    z = z_ref[...]
    zi = zi_ref[...]
    labs = lab_ref[...]
    feats = jnp.concatenate([z[:, None, :], zi], axis=1)
    labs_f = jnp.concatenate(
        [jnp.zeros((NB, 1, LABD), _f32), labs], axis=1)
    x = jnp.concatenate([feats, labs_f], axis=2)
    x_ref[...] = x
    _layer0_tail(x, w_ref, b_ref, rw_ref, h_ref, stats_ref, ib)


def _passA_kernel(xp_ref, y_ref, st_ref, w_ref, b_ref, rw_ref,
                  h_ref, x_ref, stats_ref):
    ib = pl.program_id(0)
    xp = xp_ref[...]
    y = y_ref[...]
    s = st_ref[0, :]
    t = st_ref[1, :]
    xn = _lrelu(y * s + t)
    x = jnp.concatenate([xp, xn], axis=2)
    x_ref[...] = x
    _layer0_tail(x, w_ref, b_ref, rw_ref, h_ref, stats_ref, ib)


def _layer_kernel(hp_ref, t_ref, w_ref, b_ref, rw_ref, h_ref, stats_ref):
    ib = pl.program_id(0)
    a = _lrelu(hp_ref[...] + t_ref[...])
    h = jnp.dot(a, w_ref[...], preferred_element_type=_f32) + b_ref[...]
    _accum_stats(stats_ref, h, rw_ref[...], ib)
    h_ref[...] = h.astype(_bf16)


def _proj_kernel(h3_ref, t3_ref, wl_ref, l_ref):
    a = _lrelu(h3_ref[...] + t3_ref[...])
    l_ref[...] = jnp.dot(a, wl_ref[...], preferred_element_type=_f32)


def _passE_kernel(lm_ref, x_ref, w1_ref, w2_ref, bg_ref, y_ref, stats_ref):
    ib = pl.program_id(0)
    logit = lm_ref[...]
    ii = lax.broadcasted_iota(jnp.int32, logit.shape, 1)
    jj = lax.broadcasted_iota(jnp.int32, logit.shape, 2)
    logit = logit - jnp.where(ii == jj, 1e8, 0.0)
    logit = logit - jnp.max(logit, axis=-1, keepdims=True)
    e = jnp.exp(logit)
    adj = e / jnp.sum(e, axis=-1, keepdims=True)
    xb = x_ref[...]
    d = xb.shape[-1]
    agg = lax.dot_general(adj, xb, (((2,), (1,)), ((0,), (0,))),
                          preferred_element_type=_f32)
    y = (jnp.dot(xb.reshape(NB * N, d), w1_ref[...],
                 preferred_element_type=_f32)
         + jnp.dot(agg.reshape(NB * N, d), w2_ref[...],
                   preferred_element_type=_f32)
         + bg_ref[...])
    ones = jnp.ones((NB * N, 1), _f32)
    _accum_stats(stats_ref, y, ones, ib)
    y_ref[...] = y.reshape(NB, N, 48)


def _passEf_kernel(l0_ref, x_ref, w1_ref, w2_ref, bg_ref, sig_ref, ls_ref):
    logit = l0_ref[...]
    jj = lax.broadcasted_iota(jnp.int32, logit.shape, 1)
    logit = logit - jnp.where(jj == 0, 1e8, 0.0)
    logit = logit - jnp.max(logit, axis=-1, keepdims=True)
    e = jnp.exp(logit)
    adj0 = (e / jnp.sum(e, axis=-1, keepdims=True))[:, None, :]
    xb = x_ref[...]
    d = xb.shape[-1]
    agg = lax.dot_general(adj0, xb, (((2,), (1,)), ((0,), (0,))),
                          preferred_element_type=_f32)
    lg = (jnp.dot(xb[:, 0, :], w1_ref[...], preferred_element_type=_f32)
          + jnp.dot(agg.reshape(NB, d), w2_ref[...],
                    preferred_element_type=_f32)
          + bg_ref[...])
    sig_ref[...] = 1.0 / (1.0 + jnp.exp(-lg))
    m = jnp.max(lg, axis=1, keepdims=True)
    ls_ref[...] = lg - (m + jnp.log(jnp.sum(jnp.exp(lg - m), axis=1,
                                            keepdims=True)))


def _seq_params():
    return pltpu.CompilerParams(dimension_semantics=("arbitrary",))


def _full_spec(shape):
    return pl.BlockSpec(shape, lambda ib: tuple(0 for _ in shape))


def _blk_spec(shape):
    return pl.BlockSpec(shape, lambda ib: (ib,) + tuple(0 for _ in shape[1:]))


def _bn_fold(stats, g, bt, count):
    mean = stats[0] / count
    var = stats[1] / count - mean * mean
    s = g * lax.rsqrt(var + 1e-5)
    return s, bt - mean * s


def _run_layers(h0, stats0, wp, row_w):
    h = h0
    stats = stats0
    for k in (1, 2, 3):
        s, t = _bn_fold(stats, wp['g%d' % (k - 1)], wp['bt%d' % (k - 1)],
                        float(P_PIX))
        fin = F_OUT[k - 1]
        fout = F_OUT[k]
        w = (wp['w%d' % k] * s[None, :]).T.astype(_bf16)
        tk = (t / s)[None, :].astype(_bf16)
        b = wp['b%d' % k][None, :]
        h, stats = pl.pallas_call(
            _layer_kernel,
            grid=(GRID,),
            in_specs=[_blk_spec((RB, fin)),
                      _full_spec((1, fin)),
                      _full_spec((fin, fout)),
                      _full_spec((1, fout)),
                      _full_spec((RB, 1))],
            out_specs=[_blk_spec((RB, fout)), _full_spec((2, fout))],
            out_shape=[jax.ShapeDtypeStruct((B * PPE, fout), _bf16),
                       jax.ShapeDtypeStruct((2, fout), _f32)],
            compiler_params=_seq_params(),
        )(h, tk, w, b, row_w)
    s3, t3 = _bn_fold(stats, wp['g3'], wp['bt3'], float(P_PIX))
    return h, s3, t3


def _gc_weights(gp, d):
    w1 = gp['fc_w'][:, :d].T
    w2 = gp['fc_w'][:, d:].T
    return w1, w2, gp['fc_b'][None, :]


def kernel(z, zi_s, labels_yi, params):
    zi_t = jnp.transpose(zi_s, (1, 0, 2))
    lab_t = jnp.transpose(labels_yi, (1, 0, 2))
    row_w = jnp.asarray(np.tile(_ROW_W, (NB, 1)))
    pair_idx = jnp.asarray(_PAIR_IDX)

    dims = (EMBD + LABD, EMBD + LABD + 48, EMBD + LABD + 96)
    x = None
    y_raw = None
    y_fold = None
    for r in range(3):
        wp = params['wc%d' % r] if r < 2 else params['wcl']
        gp = params['gc%d' % r] if r < 2 else params['gcl']
        d = dims[r]
        w0 = wp['w0'].T.astype(_bf16)
        b0 = wp['b0'][None, :]
        a_outs = [jax.ShapeDtypeStruct((B * PPE, F_OUT[0]), _bf16),
                  jax.ShapeDtypeStruct((B, N, d), _f32),
                  jax.ShapeDtypeStruct((2, F_OUT[0]), _f32)]
        a_ospecs = [_blk_spec((RB, F_OUT[0])),
                    _blk_spec((NB, N, d)),
                    _full_spec((2, F_OUT[0]))]
        if r == 0:
            h0, x, stats0 = pl.pallas_call(
                _passA0_kernel,
                grid=(GRID,),
                in_specs=[_blk_spec((NB, EMBD)),
                          _blk_spec((NB, S, EMBD)),
                          _blk_spec((NB, S, LABD)),
                          _full_spec((d, F_OUT[0])),
                          _full_spec((1, F_OUT[0])),
                          _full_spec((RB, 1))],
                out_specs=a_ospecs,
                out_shape=a_outs,
                compiler_params=_seq_params(),
            )(z, zi_t, lab_t, w0, b0, row_w)
        else:
            d_prev = dims[r - 1]
            h0, x, stats0 = pl.pallas_call(
                _passA_kernel,
                grid=(GRID,),
                in_specs=[_blk_spec((NB, N, d_prev)),
                          _blk_spec((NB, N, 48)),
                          _full_spec((2, 48)),
                          _full_spec((d, F_OUT[0])),
                          _full_spec((1, F_OUT[0])),
                          _full_spec((RB, 1))],
                out_specs=a_ospecs,
                out_shape=a_outs,
                compiler_params=_seq_params(),
            )(x, y_raw, y_fold, w0, b0, row_w)

        h3, s3, t3 = _run_layers(h0, stats0, wp, row_w)
        w1g, w2g, bg = _gc_weights(gp, d)

        wl_col = (wp['wl'][0] * s3)[:, None].astype(_bf16)
        t3k = (t3 / s3)[None, :].astype(_bf16)
        l_col = pl.pallas_call(
            _proj_kernel,
            grid=(GRID,),
            in_specs=[_blk_spec((RB, F_OUT[3])),
                      _full_spec((1, F_OUT[3])),
                      _full_spec((F_OUT[3], 1))],
            out_specs=_blk_spec((RB, 1)),
            out_shape=jax.ShapeDtypeStruct((B * PPE, 1), _f32),
            compiler_params=_seq_params(),
        )(h3, t3k, wl_col)
        l_pk = l_col.reshape(B, PPE)

        if r < 2:
            l_mat = jnp.take(l_pk, pair_idx, axis=1).reshape(B, N, N)
            y_raw, y_stats = pl.pallas_call(
                _passE_kernel,
                grid=(GRID,),
                in_specs=[_blk_spec((NB, N, N)),
                          _blk_spec((NB, N, d)),
                          _full_spec((d, 48)),
                          _full_spec((d, 48)),
                          _full_spec((1, 48))],
                out_specs=[_blk_spec((NB, N, 48)),
                           _full_spec((2, 48))],
                out_shape=[jax.ShapeDtypeStruct((B, N, 48), _f32),
                           jax.ShapeDtypeStruct((2, 48), _f32)],
                compiler_params=_seq_params(),
            )(l_mat, x, w1g, w2g, bg)
            ys, yt = _bn_fold(y_stats, gp['g'], gp['bt'], float(B * N))
            y_fold = jnp.stack([ys, yt], axis=0)
        else:
            l0 = jnp.take(l_pk, pair_idx[:N], axis=1)
            sig, ls = pl.pallas_call(
                _passEf_kernel,
                grid=(GRID,),
                in_specs=[_blk_spec((NB, N)),
                          _blk_spec((NB, N, d)),
                          _full_spec((d, LABD)),
                          _full_spec((d, LABD)),
                          _full_spec((1, LABD))],
                out_specs=[_blk_spec((NB, LABD)),
                           _blk_spec((NB, LABD))],
                out_shape=[jax.ShapeDtypeStruct((B, LABD), _f32),
                           jax.ShapeDtypeStruct((B, LABD), _f32)],
                compiler_params=_seq_params(),
            )(l0, x, w1g, w2g, bg)
            return sig, ls


# speedup vs baseline: 3.2669x; 1.4706x over previous
import numpy as np
import jax
import jax.numpy as jnp
from jax import lax
from jax.experimental import pallas as pl
from jax.experimental.pallas import tpu as pltpu

B = 128
S = 25
N = S + 1
NN = N * N
P_PIX = B * NN
EMBD = 128
LABD = 5
NK = 13
KP = 32
PPE = NK * KP
PB = B * PPE
EC = 8
CH = EC * PPE
NCHUNK = B // EC
NB = 32
GRID = B // NB
F_OUT = (192, 192, 96, 96)

_f32 = jnp.float32
_bf16 = jnp.bfloat16


def _pair_index_tables():
    idx = np.zeros((N, N), np.int32)
    for i in range(N):
        for j in range(N):
            if i == j:
                idx[i, j] = N
                continue
            k = (j - i) % N
            if k <= NK:
                idx[i, j] = (k - 1) * KP + i
            else:
                k2 = (i - j) % N
                idx[i, j] = (k2 - 1) * KP + j
    w = np.zeros((PPE, 1), np.float32)
    for kb in range(NK):
        w[kb * KP:kb * KP + N, 0] = 2.0 if kb < NK - 1 else 1.0
        w[kb * KP + N:(kb + 1) * KP, 0] = float(N) / float(NK * (KP - N))
    return idx.reshape(-1), np.tile(w, (EC, 1))


_PAIR_IDX, _ROW_W = _pair_index_tables()


def _lrelu(v):
    return jnp.maximum(v, 0.01 * v)


def _bn_fold_vec(stats, g, bt):
    mean = stats[0:1, :] / float(P_PIX)
    var = stats[1:2, :] / float(P_PIX) - mean * mean
    s = g * lax.rsqrt(var + 1e-5)
    return s, bt - mean * s


def _wstats(h, w):
    wh = h * w
    ps = jnp.sum(wh, axis=0)[None, :]
    pss = jnp.sum(wh * h, axis=0)[None, :]
    return jnp.concatenate([ps, pss], axis=0)


def _make_round_kernel(r, d, d_prev):
    def body(refs):
        if r == 0:
            (z_ref, zi_ref, lab_ref, rw_ref, w0_ref, b0_ref, w1_ref, b1_ref,
             w2_ref, b2_ref, w3_ref, b3_ref, g01_ref, bt01_ref, g2_ref,
             bt2_ref, g3_ref, bt3_ref, wl_ref, x_ref, l_ref,
             hbuf, qbuf) = refs
            feats = jnp.concatenate([z_ref[...][:, None, :], zi_ref[...]],
                                    axis=1)
            labs = jnp.concatenate(
                [jnp.zeros((B, 1, LABD), _f32), lab_ref[...]], axis=1)
            x_ref[...] = jnp.concatenate([feats, labs], axis=2)
        else:
            (xp_ref, y_ref, yst_ref, gg_ref, gbt_ref, rw_ref, w0_ref, b0_ref,
             w1_ref, b1_ref, w2_ref, b2_ref, w3_ref, b3_ref, g01_ref,
             bt01_ref, g2_ref, bt2_ref, g3_ref, bt3_ref, wl_ref, x_ref,
             l_ref, hbuf, qbuf) = refs
            mean = yst_ref[0:1, :] / float(B * N)
            var = yst_ref[1:2, :] / float(B * N) - mean * mean
            sy = gg_ref[...] * lax.rsqrt(var + 1e-5)
            ty = gbt_ref[...] - mean * sy
            xn = _lrelu(y_ref[...] * sy[None, :, :].reshape(1, 1, 48)
                        + ty[None, :, :].reshape(1, 1, 48))
            x_ref[...] = jnp.concatenate([xp_ref[...], xn], axis=2)

        rw = rw_ref[...]

        def l0_body(i, st):
            xs = x_ref[pl.ds(i * EC, EC), :, :].astype(_bf16)
            zpad = jnp.zeros((EC, KP - N, d), _bf16)
            xp = jnp.concatenate([xs, zpad], axis=1)
            blocks = []
            for k in range(1, NK + 1):
                sh = jnp.concatenate([xs[:, k:, :], xs[:, :k, :], zpad],
                                     axis=1)
                blocks.append(jnp.abs(xp - sh))
            a = jnp.concatenate(blocks, axis=1).reshape(CH, d)
            h = jnp.dot(a, w0_ref[...], preferred_element_type=_f32) \
                + b0_ref[...]
            hbuf[pl.ds(i * CH, CH), :] = h.astype(_bf16)
            return st + _wstats(h, rw)

        st0 = lax.fori_loop(0, NCHUNK, l0_body,
                            jnp.zeros((2, F_OUT[0]), _f32))
        s0, t0 = _bn_fold_vec(st0, g01_ref[0:1, :], bt01_ref[0:1, :])
        s0 = s0.astype(_bf16)
        t0 = t0.astype(_bf16)

        def l1_body(i, st):
            hp = hbuf[pl.ds(i * CH, CH), :]
            a = _lrelu(hp * s0 + t0)
            h = jnp.dot(a, w1_ref[...], preferred_element_type=_f32) \
                + b1_ref[...]
            hbuf[pl.ds(i * CH, CH), :] = h.astype(_bf16)
            return st + _wstats(h, rw)

        st1 = lax.fori_loop(0, NCHUNK, l1_body,
                            jnp.zeros((2, F_OUT[1]), _f32))
        s1, t1 = _bn_fold_vec(st1, g01_ref[1:2, :], bt01_ref[1:2, :])
        s1 = s1.astype(_bf16)
        t1 = t1.astype(_bf16)

        def l2_body(i, st):
            hp = hbuf[pl.ds(i * CH, CH), :]
            a = _lrelu(hp * s1 + t1)
            h = jnp.dot(a, w2_ref[...], preferred_element_type=_f32) \
                + b2_ref[...]
            qbuf[pl.ds(i * CH, CH), :] = h.astype(_bf16)
            return st + _wstats(h, rw)

        st2 = lax.fori_loop(0, NCHUNK, l2_body,
                            jnp.zeros((2, F_OUT[2]), _f32))
        s2, t2 = _bn_fold_vec(st2, g2_ref[...], bt2_ref[...])
        s2 = s2.astype(_bf16)
        t2 = t2.astype(_bf16)

        def l3_body(i, st):
            hp = qbuf[pl.ds(i * CH, CH), :]
            a = _lrelu(hp * s2 + t2)
            h = jnp.dot(a, w3_ref[...], preferred_element_type=_f32) \
                + b3_ref[...]
            qbuf[pl.ds(i * CH, CH), :] = h.astype(_bf16)
            return st + _wstats(h, rw)

        st3 = lax.fori_loop(0, NCHUNK, l3_body,
                            jnp.zeros((2, F_OUT[3]), _f32))
        s3, t3 = _bn_fold_vec(st3, g3_ref[...], bt3_ref[...])
        s3 = s3.astype(_bf16)
        t3 = t3.astype(_bf16)

        def lp_body(i, _):
            hp = qbuf[pl.ds(i * CH, CH), :]
            a = _lrelu(hp * s3 + t3)
            l = jnp.dot(a, wl_ref[...], preferred_element_type=_f32)
            l_ref[pl.ds(i * EC, EC), :] = l.reshape(EC, PPE)
            return _

        lax.fori_loop(0, NCHUNK, lp_body, 0)

    def kern(*refs):
        body(refs)

    return kern


def _passE_kernel(lm_ref, x_ref, w1_ref, w2_ref, bg_ref, y_ref, stats_ref):
    ib = pl.program_id(0)
    logit = lm_ref[...]
    ii = lax.broadcasted_iota(jnp.int32, logit.shape, 1)
    jj = lax.broadcasted_iota(jnp.int32, logit.shape, 2)
    logit = logit - jnp.where(ii == jj, 1e8, 0.0)
    logit = logit - jnp.max(logit, axis=-1, keepdims=True)
    e = jnp.exp(logit)
    adj = e / jnp.sum(e, axis=-1, keepdims=True)
    xb = x_ref[...]
    d = xb.shape[-1]
    agg = lax.dot_general(adj, xb, (((2,), (1,)), ((0,), (0,))),
                          preferred_element_type=_f32)
    y = (jnp.dot(xb.reshape(NB * N, d), w1_ref[...],
                 preferred_element_type=_f32)
         + jnp.dot(agg.reshape(NB * N, d), w2_ref[...],
                   preferred_element_type=_f32)
         + bg_ref[...])
    ps = jnp.sum(y, axis=0)[None, :]
    pss = jnp.sum(y * y, axis=0)[None, :]
    part = jnp.concatenate([ps, pss], axis=0)

    @pl.when(ib == 0)
    def _():
        stats_ref[...] = part

    @pl.when(ib > 0)
    def _():
        stats_ref[...] += part

    y_ref[...] = y.reshape(NB, N, 48)


def _passEf_kernel(l0_ref, x_ref, w1_ref, w2_ref, bg_ref, sig_ref, ls_ref):
    logit = l0_ref[...]
    jj = lax.broadcasted_iota(jnp.int32, logit.shape, 1)
    logit = logit - jnp.where(jj == 0, 1e8, 0.0)
    logit = logit - jnp.max(logit, axis=-1, keepdims=True)
    e = jnp.exp(logit)
    adj0 = (e / jnp.sum(e, axis=-1, keepdims=True))[:, None, :]
    xb = x_ref[...]
    d = xb.shape[-1]
    agg = lax.dot_general(adj0, xb, (((2,), (1,)), ((0,), (0,))),
                          preferred_element_type=_f32)
    lg = (jnp.dot(xb[:, 0, :], w1_ref[...], preferred_element_type=_f32)
          + jnp.dot(agg.reshape(NB, d), w2_ref[...],
                    preferred_element_type=_f32)
          + bg_ref[...])
    sig_ref[...] = 1.0 / (1.0 + jnp.exp(-lg))
    m = jnp.max(lg, axis=1, keepdims=True)
    ls_ref[...] = lg - (m + jnp.log(jnp.sum(jnp.exp(lg - m), axis=1,
                                            keepdims=True)))


def _seq_params():
    return pltpu.CompilerParams(dimension_semantics=("arbitrary",))


def _full_spec(shape):
    return pl.BlockSpec(shape, lambda ib: tuple(0 for _ in shape))


def _blk_spec(shape):
    return pl.BlockSpec(shape, lambda ib: (ib,) + tuple(0 for _ in shape[1:]))


def _gc_weights(gp, d):
    w1 = gp['fc_w'][:, :d].T
    w2 = gp['fc_w'][:, d:].T
    return w1, w2, gp['fc_b'][None, :]


def kernel(z, zi_s, labels_yi, params):
    zi_t = jnp.transpose(zi_s, (1, 0, 2))
    lab_t = jnp.transpose(labels_yi, (1, 0, 2))
    row_w = jnp.asarray(_ROW_W)
    pair_idx = jnp.asarray(_PAIR_IDX)

    dims = (EMBD + LABD, EMBD + LABD + 48, EMBD + LABD + 96)
    x = None
    y_raw = None
    y_stats = None
    for r in range(3):
        wp = params['wc%d' % r] if r < 2 else params['wcl']
        gp = params['gc%d' % r] if r < 2 else params['gcl']
        d = dims[r]
        d_prev = dims[r - 1] if r else None
        wrefs = [wp['w0'].T.astype(_bf16), wp['b0'][None, :],
                 wp['w1'].T.astype(_bf16), wp['b1'][None, :],
                 wp['w2'].T.astype(_bf16), wp['b2'][None, :],
                 wp['w3'].T.astype(_bf16), wp['b3'][None, :],
                 jnp.stack([wp['g0'], wp['g1']]),
                 jnp.stack([wp['bt0'], wp['bt1']]),
                 wp['g2'][None, :], wp['bt2'][None, :],
                 wp['g3'][None, :], wp['bt3'][None, :],
                 wp['wl'].T.astype(_bf16)]
        wspecs = [_full_spec(tuple(a.shape)) for a in wrefs]
        scratch = [pltpu.VMEM((PB, 192), _bf16), pltpu.VMEM((PB, 96), _bf16)]
        outs = [jax.ShapeDtypeStruct((B, N, d), _f32),
                jax.ShapeDtypeStruct((B, PPE), _f32)]
        ospecs = [_full_spec((B, N, d)), _full_spec((B, PPE))]
        if r == 0:
            ins = [z, zi_t, lab_t, row_w] + wrefs
            ispecs = [_full_spec((B, EMBD)), _full_spec((B, S, EMBD)),
                      _full_spec((B, S, LABD)), _full_spec((CH, 1))] + wspecs
        else:
            ins = [x, y_raw, y_stats, gp_prev['g'][None, :],
                   gp_prev['bt'][None, :], row_w] + wrefs
            ispecs = [_full_spec((B, N, d_prev)), _full_spec((B, N, 48)),
                      _full_spec((2, 48)), _full_spec((1, 48)),
                      _full_spec((1, 48)), _full_spec((CH, 1))] + wspecs
        x, l_pk = pl.pallas_call(
            _make_round_kernel(r, d, d_prev),
            grid=(1,),
            in_specs=ispecs,
            out_specs=ospecs,
            out_shape=outs,
            scratch_shapes=scratch,
            compiler_params=_seq_params(),
        )(*ins)

        w1g, w2g, bg = _gc_weights(gp, d)
        if r < 2:
            l_mat = jnp.take(l_pk, pair_idx, axis=1).reshape(B, N, N)
            y_raw, y_stats = pl.pallas_call(
                _passE_kernel,
                grid=(GRID,),
                in_specs=[_blk_spec((NB, N, N)),
                          _blk_spec((NB, N, d)),
                          _full_spec((d, 48)),
                          _full_spec((d, 48)),
                          _full_spec((1, 48))],
                out_specs=[_blk_spec((NB, N, 48)),
                           _full_spec((2, 48))],
                out_shape=[jax.ShapeDtypeStruct((B, N, 48), _f32),
                           jax.ShapeDtypeStruct((2, 48), _f32)],
                compiler_params=_seq_params(),
            )(l_mat, x, w1g, w2g, bg)
            gp_prev = gp
        else:
            l0 = jnp.take(l_pk, pair_idx[:N], axis=1)
            sig, ls = pl.pallas_call(
                _passEf_kernel,
                grid=(GRID,),
                in_specs=[_blk_spec((NB, N)),
                          _blk_spec((NB, N, d)),
                          _full_spec((d, LABD)),
                          _full_spec((d, LABD)),
                          _full_spec((1, LABD))],
                out_specs=[_blk_spec((NB, LABD)),
                           _blk_spec((NB, LABD))],
                out_shape=[jax.ShapeDtypeStruct((B, LABD), _f32),
                           jax.ShapeDtypeStruct((B, LABD), _f32)],
                compiler_params=_seq_params(),
            )(l0, x, w1g, w2g, bg)
            return sig, ls
